# baseline (speedup 1.0000x reference)
import functools

import jax
import jax.numpy as jnp
from jax import lax
from jax.experimental import pallas as pl
from jax.experimental.pallas import tpu as pltpu

_N_INPUT = 8
_HIDDEN = 32
_N_OUTPUT = 3
_N_ATOMS = 16
_OA = _N_OUTPUT * _N_ATOMS


def _fused_kernel(x_ref, w_ref, o_ref, *, tb, chunk):
    w0 = w_ref[0]
    w1 = w_ref[1]
    w2 = w_ref[2]
    W1 = w0[:_N_INPUT, :_HIDDEN]
    b1 = w0[_N_INPUT:_N_INPUT + 1, :_HIDDEN]
    W2 = w1[:_HIDDEN, :_HIDDEN]
    b2 = w1[_HIDDEN:_HIDDEN + 1, :_HIDDEN]
    W3 = w2[:_HIDDEN, :_OA]
    b3 = w2[_HIDDEN:_HIDDEN + 1, :_OA]
    rg = lax.broadcasted_iota(jnp.int32, (_OA, _OA), 0) // _N_ATOMS
    cg = lax.broadcasted_iota(jnp.int32, (_OA, _OA), 1) // _N_ATOMS
    G = (rg == cg).astype(jnp.float32)

    for j in range(tb // chunk):
        x = x_ref[pl.ds(j * chunk, chunk), :]
        h1 = jnp.maximum(
            jnp.dot(x, W1, preferred_element_type=jnp.float32) + b1, 0.0)
        h2 = jnp.maximum(
            jnp.dot(h1, W2, preferred_element_type=jnp.float32) + b2, 0.0)
        lg = jnp.dot(h2, W3, preferred_element_type=jnp.float32) + b3
        m = jnp.max(lg, axis=-1, keepdims=True)
        e = jnp.exp(lg - m)
        gsum = jnp.dot(e, G, preferred_element_type=jnp.float32)
        p = e * pl.reciprocal(gsum, approx=True)
        base = 3 * j * chunk
        for g in range(_N_OUTPUT):
            o_ref[0, pl.ds(base + g, chunk, 3), :] = \
                p[:, g * _N_ATOMS:(g + 1) * _N_ATOMS]


def kernel(obs, wblob):
    obs = jnp.asarray(obs, jnp.float32)
    if obs.ndim == 1:
        obs = obs[None, :]
    B = obs.shape[0]
    TB = 2048
    if B % TB != 0:
        TB = B
    CHUNK = 256 if TB % 256 == 0 else TB
    P = wblob.shape[-1]

    body = functools.partial(_fused_kernel, tb=TB, chunk=CHUNK)
    out = pl.pallas_call(
        body,
        out_shape=jax.ShapeDtypeStruct((1, B * _N_OUTPUT, _N_ATOMS),
                                       jnp.float32),
        grid=(B // TB,),
        in_specs=[
            pl.BlockSpec((TB, _N_INPUT), lambda i: (i, 0)),
            pl.BlockSpec((3, P, P), lambda i: (0, 0, 0)),
        ],
        out_specs=pl.BlockSpec((1, TB * _N_OUTPUT, _N_ATOMS),
                               lambda i: (0, i, 0)),
        compiler_params=pltpu.CompilerParams(
            dimension_semantics=("parallel",)),
    )(obs, wblob)

    return out


# speedup vs baseline: 1.3966x; 1.3966x over previous
import functools

import jax
import jax.numpy as jnp
from jax import lax
from jax.experimental import pallas as pl
from jax.experimental.pallas import tpu as pltpu

_N_INPUT = 8
_HIDDEN = 32
_N_OUTPUT = 3
_N_ATOMS = 16
_OA = _N_OUTPUT * _N_ATOMS
_AUG = _HIDDEN + 8


def _fused_t_kernel(xt_ref, w_ref, o_ref, *, tb, chunk):
    WT1 = w_ref[0][:_AUG, :16]
    WT2 = w_ref[1][:_AUG, :_AUG]
    WT3 = w_ref[2][:_OA, :_AUG]
    rg = lax.broadcasted_iota(jnp.int32, (_OA, _OA), 0) // _N_ATOMS
    cg = lax.broadcasted_iota(jnp.int32, (_OA, _OA), 1) // _N_ATOMS
    G = (rg == cg).astype(jnp.float32)

    for j in range(tb // chunk):
        xc = xt_ref[:, pl.ds(j * chunk, chunk)]
        h1 = jnp.maximum(
            jnp.dot(WT1[:, :9], xc, preferred_element_type=jnp.float32), 0.0)
        h2 = jnp.maximum(
            jnp.dot(WT2, h1, preferred_element_type=jnp.float32), 0.0)
        lt = jnp.dot(WT3, h2, preferred_element_type=jnp.float32)
        m = jnp.max(lt, axis=0, keepdims=True)
        e = jnp.exp(lt - m)
        s = jnp.dot(G, e, preferred_element_type=jnp.float32)
        p = e * pl.reciprocal(s, approx=True)
        base = 3 * j * chunk
        for g in range(_N_OUTPUT):
            pg = p[g * _N_ATOMS:(g + 1) * _N_ATOMS, :]
            o_ref[0, pl.ds(base + g, chunk, 3), :] = pg.T


def kernel(obs, wblob):
    obs = jnp.asarray(obs, jnp.float32)
    if obs.ndim == 1:
        obs = obs[None, :]
    B = obs.shape[0]
    TB = 2048
    if B % TB != 0:
        TB = B
    CHUNK = 512 if TB % 512 == 0 else TB

    xt = jnp.concatenate(
        [obs.T, jnp.ones((1, B), jnp.float32)], axis=0)
    wt = jnp.transpose(wblob, (0, 2, 1))
    P = wt.shape[-1]

    body = functools.partial(_fused_t_kernel, tb=TB, chunk=CHUNK)
    out = pl.pallas_call(
        body,
        out_shape=jax.ShapeDtypeStruct((1, B * _N_OUTPUT, _N_ATOMS),
                                       jnp.float32),
        grid=(B // TB,),
        in_specs=[
            pl.BlockSpec((_N_INPUT + 1, TB), lambda i: (0, i)),
            pl.BlockSpec((3, P, P), lambda i: (0, 0, 0)),
        ],
        out_specs=pl.BlockSpec((1, TB * _N_OUTPUT, _N_ATOMS),
                               lambda i: (0, i, 0)),
        compiler_params=pltpu.CompilerParams(
            dimension_semantics=("parallel",)),
    )(xt, wt)

    return out


# speedup vs baseline: 1.9661x; 1.4078x over previous
import functools

import jax
import jax.numpy as jnp
from jax import lax
from jax.experimental import pallas as pl
from jax.experimental.pallas import tpu as pltpu

_N_INPUT = 8
_HIDDEN = 32
_N_OUTPUT = 3
_N_ATOMS = 16
_OA = _N_OUTPUT * _N_ATOMS
_AUG = _HIDDEN + 8


def _fused_t_kernel(xt_ref, w_ref, o_ref, *, tb, chunk):
    WT1 = w_ref[0][:_AUG, :16]
    WT2 = w_ref[1][:_AUG, :_AUG]
    WT3 = w_ref[2][:_OA, :_AUG]
    rg = lax.broadcasted_iota(jnp.int32, (_OA, _OA), 0) // _N_ATOMS
    cg = lax.broadcasted_iota(jnp.int32, (_OA, _OA), 1) // _N_ATOMS
    G = (rg == cg).astype(jnp.float32)
    ri = lax.broadcasted_iota(jnp.int32, (128, 128), 0)
    ci = lax.broadcasted_iota(jnp.int32, (128, 128), 1)
    I128 = (ri == ci).astype(jnp.float32)

    for j in range(tb // chunk):
        xc = xt_ref[:, pl.ds(j * chunk, chunk)]
        h1 = jnp.maximum(
            jnp.dot(WT1[:, :9], xc, preferred_element_type=jnp.float32), 0.0)
        h2 = jnp.maximum(
            jnp.dot(WT2, h1, preferred_element_type=jnp.float32), 0.0)
        lt = jnp.dot(WT3, h2, preferred_element_type=jnp.float32)
        m = jnp.max(lt, axis=0, keepdims=True)
        e = jnp.exp(lt - m)
        s = jnp.dot(G, e, preferred_element_type=jnp.float32)
        p = e * pl.reciprocal(s, approx=True)
        base = 3 * j * chunk
        for g in range(1):
            pg = p[g * _N_ATOMS:(g + 1) * _N_ATOMS, :]
            o_ref[0, pl.ds(base + g, chunk, 3), :] = pg.T
        for g in range(1, _N_OUTPUT):
            pg = p[g * _N_ATOMS:(g + 1) * _N_ATOMS, :]
            for blk in range(chunk // 128):
                pgb = pg[:, blk * 128:(blk + 1) * 128]
                tgb = lax.dot_general(
                    I128, pgb, (((1,), (1,)), ((), ())),
                    preferred_element_type=jnp.float32)
                o_ref[0, pl.ds(base + g + 3 * 128 * blk, 128, 3), :] = tgb


def kernel(obs, wblob):
    obs = jnp.asarray(obs, jnp.float32)
    if obs.ndim == 1:
        obs = obs[None, :]
    B = obs.shape[0]
    TB = 16384
    if B % TB != 0:
        TB = B
    CHUNK = 4096 if TB % 4096 == 0 else TB

    xt = jnp.concatenate(
        [obs.T, jnp.ones((1, B), jnp.float32)], axis=0)
    wt = jnp.transpose(wblob, (0, 2, 1))
    P = wt.shape[-1]

    body = functools.partial(_fused_t_kernel, tb=TB, chunk=CHUNK)
    out = pl.pallas_call(
        body,
        out_shape=jax.ShapeDtypeStruct((1, B * _N_OUTPUT, _N_ATOMS),
                                       jnp.float32),
        grid=(B // TB,),
        in_specs=[
            pl.BlockSpec((_N_INPUT + 1, TB), lambda i: (0, i)),
            pl.BlockSpec((3, P, P), lambda i: (0, 0, 0)),
        ],
        out_specs=pl.BlockSpec((1, TB * _N_OUTPUT, _N_ATOMS),
                               lambda i: (0, i, 0)),
        compiler_params=pltpu.CompilerParams(
            dimension_semantics=("parallel",)),
    )(xt, wt)

    return out


# speedup vs baseline: 4.9874x; 2.5367x over previous
import functools

import jax
import jax.numpy as jnp
from jax import lax
from jax.experimental import pallas as pl
from jax.experimental.pallas import tpu as pltpu

_N_INPUT = 8
_HIDDEN = 32
_N_OUTPUT = 3
_N_ATOMS = 16
_OA = _N_OUTPUT * _N_ATOMS
_AUG = _HIDDEN + 8


def _fused_t_kernel(xt_ref, w_ref, o_ref, *, tb, chunk):
    WT1 = w_ref[0][:_AUG, :16]
    WT2 = w_ref[1][:_AUG, :_AUG]
    WT3 = w_ref[2][:_OA, :_AUG]
    rg = lax.broadcasted_iota(jnp.int32, (_OA, _OA), 0) // _N_ATOMS
    cg = lax.broadcasted_iota(jnp.int32, (_OA, _OA), 1) // _N_ATOMS
    G = (rg == cg).astype(jnp.float32)
    rs = lax.broadcasted_iota(jnp.int32, (384, 384), 0)
    cs = lax.broadcasted_iota(jnp.int32, (384, 384), 1)
    S = (cs == 3 * (rs % 128) + rs // 128).astype(jnp.float32)

    for j in range(tb // chunk):
        xc = xt_ref[:, pl.ds(j * chunk, chunk)]
        h1 = jnp.maximum(
            jnp.dot(WT1[:, :9], xc, preferred_element_type=jnp.float32), 0.0)
        h2 = jnp.maximum(
            jnp.dot(WT2, h1, preferred_element_type=jnp.float32), 0.0)
        lt = jnp.dot(WT3, h2, preferred_element_type=jnp.float32)
        m = jnp.max(lt, axis=0, keepdims=True)
        e = jnp.exp(lt - m)
        s = jnp.dot(G, e, preferred_element_type=jnp.float32)
        p = e * pl.reciprocal(s, approx=True)
        for blk in range(chunk // 128):
            pcat = jnp.concatenate(
                [p[g * _N_ATOMS:(g + 1) * _N_ATOMS,
                   blk * 128:(blk + 1) * 128] for g in range(_N_OUTPUT)],
                axis=1)
            tblk = jnp.dot(pcat, S,
                           preferred_element_type=jnp.float32)
            o_ref[:, pl.ds(3 * j * chunk + 384 * blk, 384)] = tblk


def kernel(obs, wblob):
    obs = jnp.asarray(obs, jnp.float32)
    if obs.ndim == 1:
        obs = obs[None, :]
    B = obs.shape[0]
    TB = 16384
    if B % TB != 0:
        TB = B
    CHUNK = 4096 if TB % 4096 == 0 else TB

    xt = jnp.concatenate(
        [obs.T, jnp.ones((1, B), jnp.float32)], axis=0)
    wt = jnp.transpose(wblob, (0, 2, 1))
    P = wt.shape[-1]

    body = functools.partial(_fused_t_kernel, tb=TB, chunk=CHUNK)
    out_t = pl.pallas_call(
        body,
        out_shape=jax.ShapeDtypeStruct((_N_ATOMS, B * _N_OUTPUT),
                                       jnp.float32),
        grid=(B // TB,),
        in_specs=[
            pl.BlockSpec((_N_INPUT + 1, TB), lambda i: (0, i)),
            pl.BlockSpec((3, P, P), lambda i: (0, 0, 0)),
        ],
        out_specs=pl.BlockSpec((_N_ATOMS, TB * _N_OUTPUT),
                               lambda i: (0, i)),
        compiler_params=pltpu.CompilerParams(
            dimension_semantics=("parallel",)),
    )(xt, wt)

    return out_t.T[None, :, :]


# speedup vs baseline: 5.6749x; 1.1378x over previous
import functools

import jax
import jax.numpy as jnp
from jax import lax
from jax.experimental import pallas as pl
from jax.experimental.pallas import tpu as pltpu

_N_INPUT = 8
_HIDDEN = 32
_N_OUTPUT = 3
_N_ATOMS = 16
_OA = _N_OUTPUT * _N_ATOMS
_AUG = _HIDDEN + 8


def _fused_t_kernel(xt_ref, w_ref, o_ref, *, tb, chunk):
    WT1 = w_ref[0][:_AUG, :16]
    b1 = WT1[:, 8:9]
    WT2 = w_ref[1][:_AUG, :_AUG]
    WT3 = w_ref[2][:_OA, :_AUG]
    rg = lax.broadcasted_iota(jnp.int32, (_OA, _OA), 0) // _N_ATOMS
    cg = lax.broadcasted_iota(jnp.int32, (_OA, _OA), 1) // _N_ATOMS
    G = (rg == cg).astype(jnp.float32)
    rs = lax.broadcasted_iota(jnp.int32, (384, 384), 0)
    cs = lax.broadcasted_iota(jnp.int32, (384, 384), 1)
    S = (cs == 3 * (rs % 128) + rs // 128).astype(jnp.float32)

    for j in range(tb // chunk):
        xc = xt_ref[:, pl.ds(j * chunk, chunk)]
        h1 = jnp.maximum(
            jnp.dot(WT1[:, :8], xc,
                    preferred_element_type=jnp.float32) + b1, 0.0)
        h2 = jnp.maximum(
            jnp.dot(WT2, h1, preferred_element_type=jnp.float32), 0.0)
        lt = jnp.dot(WT3, h2, preferred_element_type=jnp.float32)
        m = jnp.max(lt, axis=0, keepdims=True)
        e = jnp.exp(lt - m)
        s = jnp.dot(G, e, preferred_element_type=jnp.float32)
        p = e * pl.reciprocal(s, approx=True)
        for blk in range(chunk // 128):
            pcat = jnp.concatenate(
                [p[g * _N_ATOMS:(g + 1) * _N_ATOMS,
                   blk * 128:(blk + 1) * 128] for g in range(_N_OUTPUT)],
                axis=1)
            tblk = jnp.dot(pcat, S,
                           preferred_element_type=jnp.float32)
            o_ref[:, pl.ds(3 * j * chunk + 384 * blk, 384)] = tblk


def kernel(obs, wblob):
    obs = jnp.asarray(obs, jnp.float32)
    if obs.ndim == 1:
        obs = obs[None, :]
    B = obs.shape[0]
    TB = 16384
    if B % TB != 0:
        TB = B
    CHUNK = 8192 if TB % 8192 == 0 else TB

    xt = obs.T
    wt = jnp.transpose(wblob, (0, 2, 1))
    P = wt.shape[-1]

    body = functools.partial(_fused_t_kernel, tb=TB, chunk=CHUNK)
    out_t = pl.pallas_call(
        body,
        out_shape=jax.ShapeDtypeStruct((_N_ATOMS, B * _N_OUTPUT),
                                       jnp.float32),
        grid=(B // TB,),
        in_specs=[
            pl.BlockSpec((_N_INPUT, TB), lambda i: (0, i)),
            pl.BlockSpec((3, P, P), lambda i: (0, 0, 0)),
        ],
        out_specs=pl.BlockSpec((_N_ATOMS, TB * _N_OUTPUT),
                               lambda i: (0, i)),
        compiler_params=pltpu.CompilerParams(
            dimension_semantics=("parallel",)),
    )(xt, wt)

    return out_t.T[None, :, :]


# speedup vs baseline: 14.4962x; 2.5544x over previous
import functools

import jax
import jax.numpy as jnp
from jax import lax
from jax.experimental import pallas as pl
from jax.experimental.pallas import tpu as pltpu

_N_INPUT = 8
_HIDDEN = 32
_N_OUTPUT = 3
_N_ATOMS = 16
_OA = _N_OUTPUT * _N_ATOMS
_AUG = _HIDDEN + 8


def _fused_t_kernel(xt_ref, w_ref, o_ref, *, tb, chunk):
    WT1 = w_ref[0][:_AUG, :16]
    b1 = WT1[:, 8:9]
    WT2 = w_ref[1][:_AUG, :_AUG]
    WT3 = w_ref[2][:_OA, :_AUG]
    rg = lax.broadcasted_iota(jnp.int32, (_OA, _OA), 0) // _N_ATOMS
    cg = lax.broadcasted_iota(jnp.int32, (_OA, _OA), 1) // _N_ATOMS
    G = (rg == cg).astype(jnp.float32)
    rs = lax.broadcasted_iota(jnp.int32, (384, 384), 0)
    cs = lax.broadcasted_iota(jnp.int32, (384, 384), 1)
    S = (cs == 3 * (rs % 128) + rs // 128).astype(jnp.float32)

    for j in range(tb // chunk):
        xc = xt_ref[:, pl.ds(j * chunk, chunk)]
        h1 = jnp.maximum(
            jnp.dot(WT1[:, :8], xc,
                    preferred_element_type=jnp.float32) + b1, 0.0)
        h2 = jnp.maximum(
            jnp.dot(WT2, h1, preferred_element_type=jnp.float32), 0.0)
        lt = jnp.dot(WT3, h2, preferred_element_type=jnp.float32)
        m = jnp.max(lt, axis=0, keepdims=True)
        e = jnp.exp(lt - m)
        s = jnp.dot(G, e, preferred_element_type=jnp.float32)
        p = e * pl.reciprocal(s, approx=True)
        nblk = chunk // 128
        pbig = jnp.concatenate(
            [jnp.concatenate(
                [p[g * _N_ATOMS:(g + 1) * _N_ATOMS,
                   blk * 128:(blk + 1) * 128] for g in range(_N_OUTPUT)],
                axis=1) for blk in range(nblk)],
            axis=0)
        tbig = jnp.dot(pbig, S, preferred_element_type=jnp.float32)
        for blk in range(nblk):
            o_ref[:, pl.ds(3 * j * chunk + 384 * blk, 384)] = \
                tbig[blk * _N_ATOMS:(blk + 1) * _N_ATOMS, :]


def kernel(obs, wblob):
    obs = jnp.asarray(obs, jnp.float32)
    if obs.ndim == 1:
        obs = obs[None, :]
    B = obs.shape[0]
    TB = 32768
    if B % TB != 0:
        TB = B
    CHUNK = 32768 if TB % 32768 == 0 else TB

    xt = obs.T
    wt = jnp.transpose(wblob, (0, 2, 1))
    P = wt.shape[-1]

    body = functools.partial(_fused_t_kernel, tb=TB, chunk=CHUNK)
    out_t = pl.pallas_call(
        body,
        out_shape=jax.ShapeDtypeStruct((_N_ATOMS, B * _N_OUTPUT),
                                       jnp.float32),
        grid=(B // TB,),
        in_specs=[
            pl.BlockSpec((_N_INPUT, TB), lambda i: (0, i)),
            pl.BlockSpec((3, P, P), lambda i: (0, 0, 0)),
        ],
        out_specs=pl.BlockSpec((_N_ATOMS, TB * _N_OUTPUT),
                               lambda i: (0, i)),
        compiler_params=pltpu.CompilerParams(
            dimension_semantics=("parallel",)),
    )(xt, wt)

    return out_t.T[None, :, :]
